# baseline (speedup 1.0000x reference)
import functools

import jax
import jax.numpy as jnp
from jax import lax
from jax.experimental import pallas as pl
from jax.experimental.pallas import tpu as pltpu, tpu_sc as plsc

B, C, H, W = 64, 96, 32, 32
D2 = C // 2
NVEC = C // 16
ROWS = B * H
NW = 32
ROWS_PER_W = ROWS // NW
CHUNK = 8
NCHUNK = ROWS_PER_W // CHUNK


def _sc_body(x_hbm, row_hbm, col_hbm, out_hbm, rt_v, ct_v, xbuf, obuf, sem_in, sem_out):
    wid = lax.axis_index("s") * 2 + lax.axis_index("c")
    base = wid * ROWS_PER_W

    pltpu.sync_copy(row_hbm.at[pl.ds(0, H)], rt_v)
    pltpu.sync_copy(col_hbm.at[pl.ds(0, W)], ct_v)

    def chunk_body(ci, carry):
        r0 = base + ci * CHUNK
        pltpu.async_copy(x_hbm.at[pl.ds(r0, CHUNK)], xbuf, sem_in).wait()
        for rr in range(CHUNK):
            h = (r0 + rr) % H
            for w in range(W):
                for ck in range(NVEC):
                    xv = xbuf[rr, w, pl.ds(ck * 16, 16)]
                    if ck < NVEC // 2:
                        pv = ct_v[w, pl.ds(ck * 16, 16)]
                    else:
                        pv = rt_v[h, pl.ds((ck - NVEC // 2) * 16, 16)]
                    obuf[rr, w, pl.ds(ck * 16, 16)] = xv + pv
        pltpu.async_copy(obuf, out_hbm.at[pl.ds(r0, CHUNK)], sem_out).wait()
        return carry

    lax.fori_loop(0, NCHUNK, chunk_body, 0)


@jax.jit
def kernel(x, row_table, col_table):
    xt = x.transpose(0, 2, 3, 1).reshape(ROWS, W, C)
    mesh = plsc.VectorSubcoreMesh(core_axis_name="c", subcore_axis_name="s")
    k = pl.kernel(
        _sc_body,
        out_type=jax.ShapeDtypeStruct((ROWS, W, C), jnp.float32),
        mesh=mesh,
        scratch_types=[
            pltpu.VMEM((H, D2), jnp.float32),
            pltpu.VMEM((W, D2), jnp.float32),
            pltpu.VMEM((CHUNK, W, C), jnp.float32),
            pltpu.VMEM((CHUNK, W, C), jnp.float32),
            pltpu.SemaphoreType.DMA,
            pltpu.SemaphoreType.DMA,
        ],
        compiler_params=pltpu.CompilerParams(use_tc_tiling_on_sc=True),
    )
    out = k(xt, row_table, col_table)
    return out.reshape(B, H, W, C).transpose(0, 3, 1, 2)


# speedup vs baseline: 5.3847x; 5.3847x over previous
import jax
import jax.numpy as jnp
from jax.experimental import pallas as pl

B, C, H, W = 64, 96, 32, 32
D2 = C // 2


def _body(x_ref, row_ref, col_ref, out_ref):
    col_e = jnp.transpose(col_ref[:, 0:W], (1, 0))
    row_e = jnp.transpose(row_ref[:, 0:H], (1, 0))
    top = jnp.broadcast_to(col_e[None, :, :], (H, W, D2))
    bot = jnp.broadcast_to(row_e[:, None, :], (H, W, D2))
    pos = jnp.concatenate([top, bot], axis=-1)
    out_ref[...] = x_ref[...] + pos[None]


@jax.jit
def kernel(x, row_table, col_table):
    xt = x.transpose(0, 2, 3, 1)
    rt = row_table.T
    ct = col_table.T

    bblk = 16
    out = pl.pallas_call(
        _body,
        grid=(B // bblk,),
        in_specs=[
            pl.BlockSpec((bblk, H, W, C), lambda i: (i, 0, 0, 0)),
            pl.BlockSpec(rt.shape, lambda i: (0, 0)),
            pl.BlockSpec(ct.shape, lambda i: (0, 0)),
        ],
        out_specs=pl.BlockSpec((bblk, H, W, C), lambda i: (i, 0, 0, 0)),
        out_shape=jax.ShapeDtypeStruct((B, H, W, C), jnp.float32),
    )(xt, rt, ct)
    return out.transpose(0, 3, 1, 2)


# speedup vs baseline: 5.4015x; 1.0031x over previous
import jax
import jax.numpy as jnp
from jax.experimental import pallas as pl
from jax.experimental.pallas import tpu as pltpu

B, C, H, W = 64, 96, 32, 32
D2 = C // 2


def _body(x_ref, row_ref, col_ref, out_ref):
    col_e = jnp.transpose(col_ref[:, 0:W], (1, 0))
    row_e = jnp.transpose(row_ref[:, 0:H], (1, 0))
    top = jnp.broadcast_to(col_e[None, :, :], (H, W, D2))
    bot = jnp.broadcast_to(row_e[:, None, :], (H, W, D2))
    pos = jnp.concatenate([top, bot], axis=-1)
    out_ref[...] = x_ref[...] + pos[None]


@jax.jit
def kernel(x, row_table, col_table):
    xt = x.transpose(0, 2, 3, 1)
    rt = row_table.T
    ct = col_table.T

    bblk = 16
    out = pl.pallas_call(
        _body,
        grid=(B // bblk,),
        compiler_params=pltpu.CompilerParams(
            dimension_semantics=("parallel",)),
        in_specs=[
            pl.BlockSpec((bblk, H, W, C), lambda i: (i, 0, 0, 0)),
            pl.BlockSpec(rt.shape, lambda i: (0, 0)),
            pl.BlockSpec(ct.shape, lambda i: (0, 0)),
        ],
        out_specs=pl.BlockSpec((bblk, H, W, C), lambda i: (i, 0, 0, 0)),
        out_shape=jax.ShapeDtypeStruct((B, H, W, C), jnp.float32),
    )(xt, rt, ct)
    return out.transpose(0, 3, 1, 2)
